# baseline (speedup 1.0000x reference)
import jax
import jax.numpy as jnp
from jax import lax
from jax.experimental import pallas as pl
from jax.experimental.pallas import tpu as pltpu
from jax.experimental.pallas import tpu_sc as plsc

NUM_CORES = 2
NUM_SUBCORES = 16
LANES = 16
NUM_WORKERS = NUM_CORES * NUM_SUBCORES

LATENT_DIM = 32
CHUNK = 8
ROWS_PER_WORKER = 512
GROUPS = ROWS_PER_WORKER // LANES
PAIRS = GROUPS // 2


def _gmf_body(emb_u_t, emb_i_t, uidx_hbm, iidx_hbm, out_hbm,
              uidx_v, iidx_v, u_w, i_w, out_v, sem_u0, sem_i0,
              sem_u1, sem_i1):
  wid = lax.axis_index("s") * NUM_CORES + lax.axis_index("c")
  base = wid * ROWS_PER_WORKER

  pltpu.sync_copy(uidx_hbm.at[pl.ds(base, ROWS_PER_WORKER)], uidx_v)
  pltpu.sync_copy(iidx_hbm.at[pl.ds(base, ROWS_PER_WORKER)], iidx_v)

  def issue(g, buf, sem_u, sem_i):
    e0 = g * LANES
    u16 = uidx_v[pl.ds(e0, LANES)]
    i16 = iidx_v[pl.ds(e0, LANES)]
    for k in range(LANES):
      ru8 = pl.multiple_of((u16[k] // CHUNK) * CHUNK, CHUNK)
      ri8 = pl.multiple_of((i16[k] // CHUNK) * CHUNK, CHUNK)
      pltpu.async_copy(emb_u_t.at[:, pl.ds(ru8, CHUNK)], u_w.at[buf, k], sem_u)
      pltpu.async_copy(emb_i_t.at[:, pl.ds(ri8, CHUNK)], i_w.at[buf, k], sem_i)

  def drain(sem_u, sem_i):
    for _ in range(LANES):
      pltpu.make_async_copy(emb_u_t.at[:, pl.ds(0, CHUNK)],
                            u_w.at[0, 0], sem_u).wait()
      pltpu.make_async_copy(emb_i_t.at[:, pl.ds(0, CHUNK)],
                            i_w.at[0, 0], sem_i).wait()

  lane = lax.iota(jnp.int32, LANES)

  def compute(g, buf):
    e0 = g * LANES
    u16 = uidx_v[pl.ds(e0, LANES)]
    i16 = iidx_v[pl.ds(e0, LANES)]
    off_u = u16 % CHUNK
    off_i = i16 % CHUNK
    bsel = jnp.full((LANES,), buf, jnp.int32)
    acc = jnp.zeros((LANES,), jnp.float32)
    for d in range(LATENT_DIM):
      dfull = jnp.full((LANES,), d, jnp.int32)
      uv = plsc.load_gather(u_w, [bsel, lane, dfull, off_u])
      iv = plsc.load_gather(i_w, [bsel, lane, dfull, off_i])
      acc = acc + uv * iv
    rating = 1.0 / (1.0 + jnp.exp(-acc))
    out_v[pl.ds(e0, LANES)] = rating

  issue(0, 0, sem_u0, sem_i0)
  issue(1, 1, sem_u1, sem_i1)

  def pair(p, carry):
    g = 2 * p
    drain(sem_u0, sem_i0)
    compute(g, 0)

    @pl.when(p < PAIRS - 1)
    def _():
      issue(g + 2, 0, sem_u0, sem_i0)

    drain(sem_u1, sem_i1)
    compute(g + 1, 1)

    @pl.when(p < PAIRS - 1)
    def _():
      issue(g + 3, 1, sem_u1, sem_i1)

    return carry

  lax.fori_loop(0, PAIRS, pair, 0, unroll=False)

  pltpu.sync_copy(out_v, out_hbm.at[pl.ds(base, ROWS_PER_WORKER)])


@jax.jit
def _gmf(user_idx, item_idx, emb_u_t, emb_i_t):
  mesh = plsc.VectorSubcoreMesh(
      core_axis_name="c", subcore_axis_name="s",
      num_cores=NUM_CORES, num_subcores=NUM_SUBCORES)
  run = pl.kernel(
      _gmf_body,
      out_type=jax.ShapeDtypeStruct((NUM_WORKERS * ROWS_PER_WORKER,), jnp.float32),
      mesh=mesh,
      compiler_params=pltpu.CompilerParams(
          needs_layout_passes=False, use_tc_tiling_on_sc=False),
      scratch_types=[
          pltpu.VMEM((ROWS_PER_WORKER,), jnp.int32),
          pltpu.VMEM((ROWS_PER_WORKER,), jnp.int32),
          pltpu.VMEM((2, LANES, LATENT_DIM, CHUNK), jnp.float32),
          pltpu.VMEM((2, LANES, LATENT_DIM, CHUNK), jnp.float32),
          pltpu.VMEM((ROWS_PER_WORKER,), jnp.float32),
          pltpu.SemaphoreType.DMA,
          pltpu.SemaphoreType.DMA,
          pltpu.SemaphoreType.DMA,
          pltpu.SemaphoreType.DMA,
      ],
  )
  return run(emb_u_t, emb_i_t, user_idx, item_idx)


def kernel(user_indices, item_indices, emb_user, emb_item):
  batch = user_indices.shape[0]
  out = _gmf(user_indices.astype(jnp.int32), item_indices.astype(jnp.int32),
             emb_user.T, emb_item.T)
  return out.reshape(batch)


# speedup vs baseline: 5.6496x; 5.6496x over previous
import jax
import jax.numpy as jnp
from jax import lax
from jax.experimental import pallas as pl
from jax.experimental.pallas import tpu as pltpu
from jax.experimental.pallas import tpu_sc as plsc

NUM_CORES = 2
NUM_SUBCORES = 16
LANES = 16
NUM_WORKERS = NUM_CORES * NUM_SUBCORES

LATENT_DIM = 32
RPC = 4
RECW = RPC * LATENT_DIM
ROWS_PER_WORKER = 512
GROUPS = ROWS_PER_WORKER // LANES
SUB = 64
WAVES = ROWS_PER_WORKER // SUB
GPW = SUB // LANES


def _gmf_body(emb_u4, emb_i4, uidx_hbm, iidx_hbm, out_hbm,
              uidx_v, iidx_v, cu_v, ci_v, u_rec, i_rec, out_v,
              sem_u0, sem_i0, sem_u1, sem_i1):
  wid = lax.axis_index("s") * NUM_CORES + lax.axis_index("c")
  base = wid * ROWS_PER_WORKER

  pltpu.sync_copy(uidx_hbm.at[pl.ds(base, ROWS_PER_WORKER)], uidx_v)
  pltpu.sync_copy(iidx_hbm.at[pl.ds(base, ROWS_PER_WORKER)], iidx_v)

  for g in range(GROUPS):
    e0 = g * LANES
    cu_v[pl.ds(e0, LANES)] = uidx_v[pl.ds(e0, LANES)] // RPC
    ci_v[pl.ds(e0, LANES)] = iidx_v[pl.ds(e0, LANES)] // RPC

  def issue(w, buf, sem_u, sem_i):
    s = pl.ds(w * SUB, SUB)
    pltpu.async_copy(emb_u4.at[cu_v.at[s]], u_rec.at[buf], sem_u)
    pltpu.async_copy(emb_i4.at[ci_v.at[s]], i_rec.at[buf], sem_i)

  def drain(sem_u, sem_i):
    pltpu.make_async_copy(emb_u4.at[pl.ds(0, SUB)], u_rec.at[0], sem_u).wait()
    pltpu.make_async_copy(emb_i4.at[pl.ds(0, SUB)], i_rec.at[0], sem_i).wait()

  lane = lax.iota(jnp.int32, LANES)

  def compute(w, buf):
    bsel = jnp.full((LANES,), buf, jnp.int32)
    for gg in range(GPW):
      e0 = w * SUB + gg * LANES
      u16 = uidx_v[pl.ds(e0, LANES)]
      i16 = iidx_v[pl.ds(e0, LANES)]
      col_u = (u16 % RPC) * LATENT_DIM
      col_i = (i16 % RPC) * LATENT_DIM
      row = gg * LANES + lane
      acc = jnp.zeros((LANES,), jnp.float32)
      for d in range(LATENT_DIM):
        uv = plsc.load_gather(u_rec, [bsel, row, col_u + d])
        iv = plsc.load_gather(i_rec, [bsel, row, col_i + d])
        acc = acc + uv * iv
      rating = 1.0 / (1.0 + jnp.exp(-acc))
      out_v[pl.ds(e0, LANES)] = rating

  issue(0, 0, sem_u0, sem_i0)
  issue(1, 1, sem_u1, sem_i1)

  def pair(p, carry):
    w = 2 * p
    drain(sem_u0, sem_i0)
    compute(w, 0)

    @pl.when(p < WAVES // 2 - 1)
    def _():
      issue(w + 2, 0, sem_u0, sem_i0)

    drain(sem_u1, sem_i1)
    compute(w + 1, 1)

    @pl.when(p < WAVES // 2 - 1)
    def _():
      issue(w + 3, 1, sem_u1, sem_i1)

    return carry

  lax.fori_loop(0, WAVES // 2, pair, 0, unroll=False)

  pltpu.sync_copy(out_v, out_hbm.at[pl.ds(base, ROWS_PER_WORKER)])


@jax.jit
def _gmf(user_idx, item_idx, emb_u4, emb_i4):
  mesh = plsc.VectorSubcoreMesh(
      core_axis_name="c", subcore_axis_name="s",
      num_cores=NUM_CORES, num_subcores=NUM_SUBCORES)
  run = pl.kernel(
      _gmf_body,
      out_type=jax.ShapeDtypeStruct((NUM_WORKERS * ROWS_PER_WORKER,), jnp.float32),
      mesh=mesh,
      compiler_params=pltpu.CompilerParams(
          needs_layout_passes=False, use_tc_tiling_on_sc=False),
      scratch_types=[
          pltpu.VMEM((ROWS_PER_WORKER,), jnp.int32),
          pltpu.VMEM((ROWS_PER_WORKER,), jnp.int32),
          pltpu.VMEM((ROWS_PER_WORKER,), jnp.int32),
          pltpu.VMEM((ROWS_PER_WORKER,), jnp.int32),
          pltpu.VMEM((2, SUB, RECW), jnp.float32),
          pltpu.VMEM((2, SUB, RECW), jnp.float32),
          pltpu.VMEM((ROWS_PER_WORKER,), jnp.float32),
          pltpu.SemaphoreType.DMA,
          pltpu.SemaphoreType.DMA,
          pltpu.SemaphoreType.DMA,
          pltpu.SemaphoreType.DMA,
      ],
  )
  return run(emb_u4, emb_i4, user_idx, item_idx)


def kernel(user_indices, item_indices, emb_user, emb_item):
  batch = user_indices.shape[0]
  nrec = emb_user.shape[0] // RPC
  out = _gmf(user_indices.astype(jnp.int32), item_indices.astype(jnp.int32),
             emb_user.reshape(nrec, RECW), emb_item.reshape(nrec, RECW))
  return out.reshape(batch)
